# baseline (speedup 1.0000x reference)
import functools

import jax
import jax.numpy as jnp
from jax import lax
from jax.experimental import pallas as pl
from jax.experimental.pallas import tpu as pltpu
from jax.experimental.pallas import tpu_sc as plsc

N = 10000
E = 320000
D_IN = 128
D_HID = 256
D_OUT = 64

NC = 2
NS = 16
LN = 16
NW = NC * NS
NP = 10240
RPS = NP // NS
ET = E // NW
CH = 125
NCH = ET // CH
ZR = 16
SBC = 16
NSB = NCH // SBC
QW = SBC * CH
RB = 1024

_MESH = plsc.VectorSubcoreMesh(
    core_axis_name="c", subcore_axis_name="s", num_cores=NC, num_subcores=NS)
_SC_PARAMS = pltpu.CompilerParams(needs_layout_passes=False)


def _full(v):
    return jnp.full((LN,), v, dtype=jnp.int32)


def _deg_body(dstf, ewf, out, dst_v, ew_v, deg_v, red16_v, red_v, stage_sh):
    c = lax.axis_index("c")
    s = lax.axis_index("s")
    w = c * NS + s
    pltpu.sync_copy(dstf.at[pl.ds(w * ET, ET)], dst_v)
    pltpu.sync_copy(ewf.at[pl.ds(w * ET, ET)], ew_v)

    def zero(i, _):
        deg_v[pl.ds(i * LN, LN)] = jnp.zeros((LN,), jnp.float32)
        return _
    lax.fori_loop(0, NP // LN, zero, None)

    def grp(g, _):
        dst16 = dst_v[pl.ds(g * LN, LN)]
        ew16 = ew_v[pl.ds(g * LN, LN)]
        plsc.addupdate_scatter(deg_v, [dst16], ew16)
        return _
    lax.fori_loop(0, ET // LN, grp, None)

    pltpu.sync_copy(deg_v, stage_sh.at[s])
    plsc.subcore_barrier()

    def rcp(r, _):
        pltpu.sync_copy(stage_sh.at[r].at[pl.ds(s * RPS, RPS)], red16_v.at[r])
        return _
    lax.fori_loop(0, NS, rcp, None)

    def rsum(t, _):
        acc = jnp.zeros((LN,), jnp.float32)
        for r in range(NS):
            acc = acc + red16_v[r, pl.ds(t * LN, LN)]
        red_v[pl.ds(t * LN, LN)] = acc
        return _
    lax.fori_loop(0, RPS // LN, rsum, None)
    pltpu.sync_copy(red_v, out.at[c].at[pl.ds(s * RPS, RPS)])


_deg_call = functools.partial(
    pl.kernel,
    out_type=jax.ShapeDtypeStruct((NC, NP), jnp.float32),
    mesh=_MESH,
    scratch_types=[
        pltpu.VMEM((ET,), jnp.int32),
        pltpu.VMEM((ET,), jnp.float32),
        pltpu.VMEM((NP,), jnp.float32),
        pltpu.VMEM((NS, RPS), jnp.float32),
        pltpu.VMEM((RPS,), jnp.float32),
        pltpu.VMEM_SHARED((NS, NP), jnp.float32),
    ],
    compiler_params=_SC_PARAMS,
)(_deg_body)


def _prop_body(src3, dst3, ewf, xs, out,
               sq_v, dq_v, eq_v, rows0_v, rows1_v, zb_v, acc_sh,
               gsem0, gsem1, ssem0, ssem1, qsem):
    rbufs = (rows0_v, rows1_v)
    ssems = (ssem0, ssem1)
    c = lax.axis_index("c")
    s = lax.axis_index("s")
    w = c * NS + s
    gsems = (gsem0, gsem1)

    def stage_issue(q, slot):
        pltpu.async_copy(src3.at[w].at[pl.ds(q * SBC, SBC)],
                         sq_v.at[slot], qsem)
        pltpu.async_copy(dst3.at[w].at[pl.ds(q * SBC, SBC)],
                         dq_v.at[slot], qsem)
        pltpu.async_copy(ewf.at[w * NSB + q], eq_v.at[slot], qsem)

    def stage_wait(slot):
        pltpu.make_async_copy(src3.at[w].at[pl.ds(0, SBC)],
                              sq_v.at[slot], qsem).wait()
        pltpu.make_async_copy(dst3.at[w].at[pl.ds(0, SBC)],
                              dq_v.at[slot], qsem).wait()
        pltpu.make_async_copy(ewf.at[w * NSB], eq_v.at[slot], qsem).wait()

    def gather_issue(j, b):
        slot = (j // SBC) % 2
        jj = j % SBC
        pltpu.async_copy(xs.at[sq_v.at[slot, jj]], rbufs[b], gsems[b])

    stage_issue(0, 0)

    def zrow(i, _):
        for k in range(D_IN // LN):
            zb_v[i, pl.ds(k * LN, LN)] = jnp.zeros((LN,), jnp.float32)
        return _
    lax.fori_loop(0, ZR, zrow, None)
    stage_wait(0)
    gather_issue(0, 0)

    def zcp(t, _):
        pltpu.sync_copy(zb_v, acc_sh.at[pl.ds(s * RPS + t * ZR, ZR)])
        return _
    lax.fori_loop(0, RPS // ZR, zcp, None)
    plsc.subcore_barrier()

    def pair(p, _):
        for b in range(2):
            j = 2 * p + b
            slot = (j // SBC) % 2
            jj = j % SBC
            pltpu.make_async_copy(xs.at[sq_v.at[slot, jj]],
                                  rbufs[b], gsems[b]).wait()
            @pl.when(jnp.logical_and(jj == 0, j + SBC < NCH))
            def _st():
                stage_issue(j // SBC + 1, 1 - slot)
            @pl.when(jnp.logical_and(jj == SBC - 1, j + 1 < NCH))
            def _sw():
                stage_wait(1 - slot)

            @pl.when(j >= 1)
            def _w():
                pltpu.make_async_copy(rbufs[1 - b],
                                      acc_sh.at[dq_v.at[slot, jj]],
                                      ssems[1 - b]).wait()

            @pl.when(j + 1 < NCH)
            def _g():
                gather_issue(j + 1, 1 - b)
            base = jj * CH
            rb = rbufs[b]

            @plsc.parallel_loop(0, CH, step=1, unroll=5)
            def srow(i):
                ewb = plsc.load_gather(eq_v, [_full(slot), _full(base) + _full(i)])
                for k in range(D_IN // LN):
                    sl = pl.ds(k * LN, LN)
                    rb[i, sl] = rb[i, sl] * ewb
            pltpu.async_copy(rb, acc_sh.at[dq_v.at[slot, jj]],
                             ssems[b], add=True)
        return _
    lax.fori_loop(0, NCH // 2, pair, None)
    pltpu.make_async_copy(rows1_v,
                          acc_sh.at[dq_v.at[(NCH // SBC - 1) % 2, SBC - 1]],
                          ssems[1]).wait()
    plsc.subcore_barrier()
    pltpu.sync_copy(acc_sh.at[pl.ds(s * RPS, RPS)],
                    out.at[c].at[pl.ds(s * RPS, RPS)])


_prop_call = functools.partial(
    pl.kernel,
    out_type=jax.ShapeDtypeStruct((NC, NP, D_IN), jnp.float32),
    mesh=_MESH,
    scratch_types=[
        pltpu.VMEM((2, SBC, CH), jnp.int32),
        pltpu.VMEM((2, SBC, CH), jnp.int32),
        pltpu.VMEM((2, QW), jnp.float32),
        pltpu.VMEM((CH, D_IN), jnp.float32),
        pltpu.VMEM((CH, D_IN), jnp.float32),
        pltpu.VMEM((ZR, D_IN), jnp.float32),
        pltpu.VMEM_SHARED((NP, D_IN), jnp.float32),
        pltpu.SemaphoreType.DMA,
        pltpu.SemaphoreType.DMA,
        pltpu.SemaphoreType.DMA,
        pltpu.SemaphoreType.DMA,
        pltpu.SemaphoreType.DMA,
    ],
    compiler_params=_SC_PARAMS,
)(_prop_body)


def _prep_body(degp_ref, xp_ref, dis_ref, xs_ref):
    deg = degp_ref[0, :, 0:1] + degp_ref[1, :, 0:1]
    dis = jnp.where(deg > 0, lax.rsqrt(deg), 0.0)
    dis_ref[...] = dis
    xs_ref[...] = xp_ref[...] * dis


_prep_call = pl.pallas_call(
    _prep_body,
    grid=(NP // RB,),
    in_specs=[
        pl.BlockSpec((NC, RB, 1), lambda j: (0, j, 0)),
        pl.BlockSpec((RB, D_IN), lambda j: (j, 0)),
    ],
    out_specs=[
        pl.BlockSpec((RB, 1), lambda j: (j, 0)),
        pl.BlockSpec((RB, D_IN), lambda j: (j, 0)),
    ],
    out_shape=[
        jax.ShapeDtypeStruct((NP, 1), jnp.float32),
        jax.ShapeDtypeStruct((NP, D_IN), jnp.float32),
    ],
)


def _mid_body(acc_ref, dis_ref, w0_ref, b0_ref, wc_ref, hcs_ref):
    dis = dis_ref[...]
    a = (acc_ref[0] + acc_ref[1]) * dis
    h = jnp.dot(a, w0_ref[...], preferred_element_type=jnp.float32)
    h = jnp.maximum(h + b0_ref[...], 0.0)
    hc = jnp.dot(h, wc_ref[...], preferred_element_type=jnp.float32)
    hcs_ref[...] = hc * dis


_mid_call = pl.pallas_call(
    _mid_body,
    grid=(NP // RB,),
    in_specs=[
        pl.BlockSpec((NC, RB, D_IN), lambda j: (0, j, 0)),
        pl.BlockSpec((RB, 1), lambda j: (j, 0)),
        pl.BlockSpec((D_IN, D_HID), lambda j: (0, 0)),
        pl.BlockSpec((1, D_HID), lambda j: (0, 0)),
        pl.BlockSpec((D_HID, D_IN), lambda j: (0, 0)),
    ],
    out_specs=pl.BlockSpec((RB, D_IN), lambda j: (j, 0)),
    out_shape=jax.ShapeDtypeStruct((NP, D_IN), jnp.float32),
)


def _post_body(acc_ref, dis_ref, bm_ref, bl_ref, mean_ref, lv_ref):
    m = (acc_ref[0] + acc_ref[1]) * dis_ref[...]
    mean_ref[...] = m[:, :D_OUT] + bm_ref[...]
    lv_ref[...] = m[:, D_OUT:] + bl_ref[...]


_post_call = pl.pallas_call(
    _post_body,
    grid=(NP // RB,),
    in_specs=[
        pl.BlockSpec((NC, RB, D_IN), lambda j: (0, j, 0)),
        pl.BlockSpec((RB, 1), lambda j: (j, 0)),
        pl.BlockSpec((1, D_OUT), lambda j: (0, 0)),
        pl.BlockSpec((1, D_OUT), lambda j: (0, 0)),
    ],
    out_specs=[
        pl.BlockSpec((RB, D_OUT), lambda j: (j, 0)),
        pl.BlockSpec((RB, D_OUT), lambda j: (j, 0)),
    ],
    out_shape=[
        jax.ShapeDtypeStruct((NP, D_OUT), jnp.float32),
        jax.ShapeDtypeStruct((NP, D_OUT), jnp.float32),
    ],
)


def kernel(x, edge_index, edge_weight, W0, b0, Wm, bm, Wl, bl):
    src3 = edge_index[0].reshape(NW, NCH, CH)
    dst3 = edge_index[1].reshape(NW, NCH, CH)
    xp = jnp.pad(x, ((0, NP - N), (0, 0)))

    degp = _deg_call(edge_index[1], edge_weight)
    dis, xs = _prep_call(degp.reshape(NC, NP, 1), xp)
    ew2 = edge_weight.reshape(NW * NSB, QW)
    acc1 = _prop_call(src3, dst3, ew2, xs)
    hcs = _mid_call(acc1, dis, W0, b0.reshape(1, D_HID),
                    jnp.concatenate([Wm, Wl], axis=1))
    acc2 = _prop_call(src3, dst3, ew2, hcs)
    mean, logvar = _post_call(acc2, dis, bm.reshape(1, D_OUT),
                              bl.reshape(1, D_OUT))
    return mean[:N], logvar[:N]


# speedup vs baseline: 1.0208x; 1.0208x over previous
import functools

import jax
import jax.numpy as jnp
from jax import lax
from jax.experimental import pallas as pl
from jax.experimental.pallas import tpu as pltpu
from jax.experimental.pallas import tpu_sc as plsc

N = 10000
E = 320000
D_IN = 128
D_HID = 256
D_OUT = 64

NC = 2
NS = 16
LN = 16
NW = NC * NS
NP = 10240
RPS = NP // NS
ET = E // NW
CH = 125
NCH = ET // CH
ZR = 16
SBC = 16
NSB = NCH // SBC
QW = SBC * CH
RB = 1024

_MESH = plsc.VectorSubcoreMesh(
    core_axis_name="c", subcore_axis_name="s", num_cores=NC, num_subcores=NS)
_SC_PARAMS = pltpu.CompilerParams(needs_layout_passes=False)


def _full(v):
    return jnp.full((LN,), v, dtype=jnp.int32)


def _deg_body(dstf, ewf, out, dst_v, ew_v, deg_v, red16_v, red_v, stage_sh):
    c = lax.axis_index("c")
    s = lax.axis_index("s")
    w = c * NS + s
    pltpu.sync_copy(dstf.at[pl.ds(w * ET, ET)], dst_v)
    pltpu.sync_copy(ewf.at[pl.ds(w * ET, ET)], ew_v)

    def zero(i, _):
        deg_v[pl.ds(i * LN, LN)] = jnp.zeros((LN,), jnp.float32)
        return _
    lax.fori_loop(0, NP // LN, zero, None)

    def grp(g, _):
        dst16 = dst_v[pl.ds(g * LN, LN)]
        ew16 = ew_v[pl.ds(g * LN, LN)]
        plsc.addupdate_scatter(deg_v, [dst16], ew16)
        return _
    lax.fori_loop(0, ET // LN, grp, None)

    pltpu.sync_copy(deg_v, stage_sh.at[s])
    plsc.subcore_barrier()

    def rcp(r, _):
        pltpu.sync_copy(stage_sh.at[r].at[pl.ds(s * RPS, RPS)], red16_v.at[r])
        return _
    lax.fori_loop(0, NS, rcp, None)

    def rsum(t, _):
        acc = jnp.zeros((LN,), jnp.float32)
        for r in range(NS):
            acc = acc + red16_v[r, pl.ds(t * LN, LN)]
        red_v[pl.ds(t * LN, LN)] = acc
        return _
    lax.fori_loop(0, RPS // LN, rsum, None)
    pltpu.sync_copy(red_v, out.at[c].at[pl.ds(s * RPS, RPS)])


_deg_call = functools.partial(
    pl.kernel,
    out_type=jax.ShapeDtypeStruct((NC, NP), jnp.float32),
    mesh=_MESH,
    scratch_types=[
        pltpu.VMEM((ET,), jnp.int32),
        pltpu.VMEM((ET,), jnp.float32),
        pltpu.VMEM((NP,), jnp.float32),
        pltpu.VMEM((NS, RPS), jnp.float32),
        pltpu.VMEM((RPS,), jnp.float32),
        pltpu.VMEM_SHARED((NS, NP), jnp.float32),
    ],
    compiler_params=_SC_PARAMS,
)(_deg_body)


def _prop_body(src3, dst3, ewf, xs, out,
               sq_v, dq_v, eq_v, rows0_v, rows1_v, zb_v, acc_sh,
               gsem0, gsem1, ssem0, ssem1, qsem):
    rbufs = (rows0_v, rows1_v)
    ssems = (ssem0, ssem1)
    c = lax.axis_index("c")
    s = lax.axis_index("s")
    w = c * NS + s
    gsems = (gsem0, gsem1)

    def stage_issue(q, slot):
        pltpu.async_copy(src3.at[w].at[pl.ds(q * SBC, SBC)],
                         sq_v.at[slot], qsem)
        pltpu.async_copy(dst3.at[w].at[pl.ds(q * SBC, SBC)],
                         dq_v.at[slot], qsem)
        pltpu.async_copy(ewf.at[w * NSB + q], eq_v.at[slot], qsem)

    def stage_wait(slot):
        pltpu.make_async_copy(src3.at[w].at[pl.ds(0, SBC)],
                              sq_v.at[slot], qsem).wait()
        pltpu.make_async_copy(dst3.at[w].at[pl.ds(0, SBC)],
                              dq_v.at[slot], qsem).wait()
        pltpu.make_async_copy(ewf.at[w * NSB], eq_v.at[slot], qsem).wait()

    def gather_issue(j, b):
        slot = (j // SBC) % 2
        jj = j % SBC
        pltpu.async_copy(xs.at[sq_v.at[slot, jj]], rbufs[b], gsems[b])

    stage_issue(0, 0)

    def zrow(i, _):
        for k in range(D_IN // LN):
            zb_v[i, pl.ds(k * LN, LN)] = jnp.zeros((LN,), jnp.float32)
        return _
    lax.fori_loop(0, ZR, zrow, None)
    stage_wait(0)
    gather_issue(0, 0)

    def zcp(t, _):
        pltpu.sync_copy(zb_v, acc_sh.at[pl.ds(s * RPS + t * ZR, ZR)])
        return _
    lax.fori_loop(0, RPS // ZR, zcp, None)
    plsc.subcore_barrier()

    def pair(p, _):
        for b in range(2):
            j = 2 * p + b
            slot = (j // SBC) % 2
            jj = j % SBC
            @pl.when(j >= 1)
            def _w():
                pltpu.make_async_copy(rbufs[1 - b],
                                      acc_sh.at[dq_v.at[slot, jj]],
                                      ssems[1 - b]).wait()
            @pl.when(jnp.logical_and(jj == 0, j + SBC < NCH))
            def _st():
                stage_issue(j // SBC + 1, 1 - slot)
            @pl.when(jnp.logical_and(jj == SBC - 1, j + 1 < NCH))
            def _sw():
                stage_wait(1 - slot)

            @pl.when(j + 1 < NCH)
            def _g():
                gather_issue(j + 1, 1 - b)
            pltpu.make_async_copy(xs.at[sq_v.at[slot, jj]],
                                  rbufs[b], gsems[b]).wait()
            base = jj * CH
            rb = rbufs[b]

            @plsc.parallel_loop(0, CH, step=1, unroll=5)
            def srow(i):
                ewb = plsc.load_gather(eq_v, [_full(slot), _full(base) + _full(i)])
                for k in range(D_IN // LN):
                    sl = pl.ds(k * LN, LN)
                    rb[i, sl] = rb[i, sl] * ewb
            pltpu.async_copy(rb, acc_sh.at[dq_v.at[slot, jj]],
                             ssems[b], add=True)
        return _
    lax.fori_loop(0, NCH // 2, pair, None)
    pltpu.make_async_copy(rows1_v,
                          acc_sh.at[dq_v.at[(NCH // SBC - 1) % 2, SBC - 1]],
                          ssems[1]).wait()
    plsc.subcore_barrier()
    pltpu.sync_copy(acc_sh.at[pl.ds(s * RPS, RPS)],
                    out.at[c].at[pl.ds(s * RPS, RPS)])


_prop_call = functools.partial(
    pl.kernel,
    out_type=jax.ShapeDtypeStruct((NC, NP, D_IN), jnp.float32),
    mesh=_MESH,
    scratch_types=[
        pltpu.VMEM((2, SBC, CH), jnp.int32),
        pltpu.VMEM((2, SBC, CH), jnp.int32),
        pltpu.VMEM((2, QW), jnp.float32),
        pltpu.VMEM((CH, D_IN), jnp.float32),
        pltpu.VMEM((CH, D_IN), jnp.float32),
        pltpu.VMEM((ZR, D_IN), jnp.float32),
        pltpu.VMEM_SHARED((NP, D_IN), jnp.float32),
        pltpu.SemaphoreType.DMA,
        pltpu.SemaphoreType.DMA,
        pltpu.SemaphoreType.DMA,
        pltpu.SemaphoreType.DMA,
        pltpu.SemaphoreType.DMA,
    ],
    compiler_params=_SC_PARAMS,
)(_prop_body)


def _prep_body(degp_ref, xp_ref, dis_ref, xs_ref):
    deg = degp_ref[0, :, 0:1] + degp_ref[1, :, 0:1]
    dis = jnp.where(deg > 0, lax.rsqrt(deg), 0.0)
    dis_ref[...] = dis
    xs_ref[...] = xp_ref[...] * dis


_prep_call = pl.pallas_call(
    _prep_body,
    grid=(NP // RB,),
    in_specs=[
        pl.BlockSpec((NC, RB, 1), lambda j: (0, j, 0)),
        pl.BlockSpec((RB, D_IN), lambda j: (j, 0)),
    ],
    out_specs=[
        pl.BlockSpec((RB, 1), lambda j: (j, 0)),
        pl.BlockSpec((RB, D_IN), lambda j: (j, 0)),
    ],
    out_shape=[
        jax.ShapeDtypeStruct((NP, 1), jnp.float32),
        jax.ShapeDtypeStruct((NP, D_IN), jnp.float32),
    ],
)


def _mid_body(acc_ref, dis_ref, w0_ref, b0_ref, wc_ref, hcs_ref):
    dis = dis_ref[...]
    a = (acc_ref[0] + acc_ref[1]) * dis
    h = jnp.dot(a, w0_ref[...], preferred_element_type=jnp.float32)
    h = jnp.maximum(h + b0_ref[...], 0.0)
    hc = jnp.dot(h, wc_ref[...], preferred_element_type=jnp.float32)
    hcs_ref[...] = hc * dis


_mid_call = pl.pallas_call(
    _mid_body,
    grid=(NP // RB,),
    in_specs=[
        pl.BlockSpec((NC, RB, D_IN), lambda j: (0, j, 0)),
        pl.BlockSpec((RB, 1), lambda j: (j, 0)),
        pl.BlockSpec((D_IN, D_HID), lambda j: (0, 0)),
        pl.BlockSpec((1, D_HID), lambda j: (0, 0)),
        pl.BlockSpec((D_HID, D_IN), lambda j: (0, 0)),
    ],
    out_specs=pl.BlockSpec((RB, D_IN), lambda j: (j, 0)),
    out_shape=jax.ShapeDtypeStruct((NP, D_IN), jnp.float32),
)


def _post_body(acc_ref, dis_ref, bm_ref, bl_ref, mean_ref, lv_ref):
    m = (acc_ref[0] + acc_ref[1]) * dis_ref[...]
    mean_ref[...] = m[:, :D_OUT] + bm_ref[...]
    lv_ref[...] = m[:, D_OUT:] + bl_ref[...]


_post_call = pl.pallas_call(
    _post_body,
    grid=(NP // RB,),
    in_specs=[
        pl.BlockSpec((NC, RB, D_IN), lambda j: (0, j, 0)),
        pl.BlockSpec((RB, 1), lambda j: (j, 0)),
        pl.BlockSpec((1, D_OUT), lambda j: (0, 0)),
        pl.BlockSpec((1, D_OUT), lambda j: (0, 0)),
    ],
    out_specs=[
        pl.BlockSpec((RB, D_OUT), lambda j: (j, 0)),
        pl.BlockSpec((RB, D_OUT), lambda j: (j, 0)),
    ],
    out_shape=[
        jax.ShapeDtypeStruct((NP, D_OUT), jnp.float32),
        jax.ShapeDtypeStruct((NP, D_OUT), jnp.float32),
    ],
)


def kernel(x, edge_index, edge_weight, W0, b0, Wm, bm, Wl, bl):
    src3 = edge_index[0].reshape(NW, NCH, CH)
    dst3 = edge_index[1].reshape(NW, NCH, CH)
    xp = jnp.pad(x, ((0, NP - N), (0, 0)))

    degp = _deg_call(edge_index[1], edge_weight)
    dis, xs = _prep_call(degp.reshape(NC, NP, 1), xp)
    ew2 = edge_weight.reshape(NW * NSB, QW)
    acc1 = _prop_call(src3, dst3, ew2, xs)
    hcs = _mid_call(acc1, dis, W0, b0.reshape(1, D_HID),
                    jnp.concatenate([Wm, Wl], axis=1))
    acc2 = _prop_call(src3, dst3, ew2, hcs)
    mean, logvar = _post_call(acc2, dis, bm.reshape(1, D_OUT),
                              bl.reshape(1, D_OUT))
    return mean[:N], logvar[:N]
